# baseline (speedup 1.0000x reference)
import jax
import jax.numpy as jnp
from jax import lax
from jax.experimental import pallas as pl
from jax.experimental.pallas import tpu as pltpu
from jax.experimental.pallas import tpu_sc as plsc

_K = 99
_PAD_K = 128
_EPS = 1e-05
_D = 26
_B = 16384
_NW = 32
_ROWS = _B // _NW
_CHUNK = _ROWS * _D
_VECS = _CHUNK // 16
_PERIOD = 13


def _body(x_hbm, q_hbm, p_hbm, m_hbm, s_hbm, out_hbm,
          x_v, o_v, q_v, p_v, m_v, s_v, qpad_v,
          patr_v, patq_v, pati_v, pate_v, sem):
    wid = lax.axis_index("s") * 2 + lax.axis_index("c")
    rbase = wid * _ROWS

    xcopy = pltpu.async_copy(x_hbm.at[pl.ds(rbase, _ROWS), :], x_v, sem)
    pltpu.sync_copy(q_hbm, q_v)
    pltpu.sync_copy(p_hbm, p_v)
    pltpu.sync_copy(m_hbm, m_v)
    pltpu.sync_copy(s_hbm, s_v)

    @plsc.parallel_loop(0, _D * _PAD_K // 16, step=1, unroll=4)
    def build(j):
        flat = j * 16 + lax.iota(jnp.int32, 16)
        d = lax.shift_right_logical(flat, 7)
        c = lax.bitwise_and(flat, _PAD_K - 1)
        cc = jnp.minimum(c, _K - 1)
        qv = plsc.load_gather(q_v, [d, cc])
        sv = plsc.load_gather(s_v, [d])
        mv = plsc.load_gather(m_v, [d])
        qpad_v[pl.ds(j * 16, 16)] = jnp.where(c > _K - 1, jnp.inf, qv * sv + mv)

    for j in range(_PERIOD):
        lane = lax.iota(jnp.int32, 16) + (j * 16)
        dd = lane % _D
        sv = plsc.load_gather(s_v, [dd])
        patq_v[pl.ds(j * 16, 16)] = dd * _PAD_K
        patr_v[pl.ds(j * 16, 16)] = lax.div(lane, _D)
        pati_v[pl.ds(j * 16, 16)] = 1.0 / sv
        pate_v[pl.ds(j * 16, 16)] = _EPS * sv

    xcopy.wait()

    @plsc.parallel_loop(0, _VECS // _PERIOD, step=1, unroll=2)
    def body(g):
        for j in range(_PERIOD):
            poff = j * 16
            qb = patq_v[pl.ds(poff, 16)]
            rl = patr_v[pl.ds(poff, 16)] + g * 8
            cl = lax.shift_right_logical(qb, 7)
            iv = pati_v[pl.ds(poff, 16)]
            es = pate_v[pl.ds(poff, 16)]
            xv = plsc.load_gather(x_v, [rl, cl])

            apos = qb
            for step in (64, 32, 16, 8, 4, 2, 1):
                qv = plsc.load_gather(qpad_v, [apos + (step - 1)])
                apos = jnp.where(qv < xv, apos + step, apos)
            idx = apos - qb

            left = jnp.maximum(idx - 1, 0)
            right = jnp.minimum(idx, _K - 1)
            qls = plsc.load_gather(qpad_v, [qb + left])
            qrs = plsc.load_gather(qpad_v, [qb + right])
            pL = plsc.load_gather(p_v, [left])
            pR = plsc.load_gather(p_v, [right])

            res = pL + (xv - qls) * (pR - pL) / (qrs - qls + es)
            mlow = (idx == 0) & (xv < qls)
            mhigh = (idx == _K) & (xv > qrs)
            ta = jnp.where(mlow, qls - xv, xv - qrs) * iv
            e = jnp.exp(-2.0 * jnp.maximum(ta, 0.0))
            th = (1.0 - e) / (1.0 + e)
            res = jnp.where(mlow, pL - pL * th, res)
            res = jnp.where(mhigh, pR + (1.0 - pR) * th, res)
            plsc.store_scatter(o_v, [rl, cl], res)

    pltpu.sync_copy(o_v, out_hbm.at[pl.ds(rbase, _ROWS), :])


@jax.jit
def _qnorm(x, quantiles, probs, initial_means, initial_stds):
    mesh = plsc.VectorSubcoreMesh(core_axis_name="c", subcore_axis_name="s")
    f = pl.kernel(
        _body,
        out_type=jax.ShapeDtypeStruct((_B, _D), jnp.float32),
        mesh=mesh,
        compiler_params=pltpu.CompilerParams(
            needs_layout_passes=False, use_tc_tiling_on_sc=False),
        scratch_types=[
            pltpu.VMEM((_ROWS, _D), jnp.float32),
            pltpu.VMEM((_ROWS, _D), jnp.float32),
            pltpu.VMEM((_D, _K), jnp.float32),
            pltpu.VMEM((_K,), jnp.float32),
            pltpu.VMEM((_D,), jnp.float32),
            pltpu.VMEM((_D,), jnp.float32),
            pltpu.VMEM((_D * _PAD_K,), jnp.float32),
            pltpu.VMEM((16 * _PERIOD,), jnp.int32),
            pltpu.VMEM((16 * _PERIOD,), jnp.int32),
            pltpu.VMEM((16 * _PERIOD,), jnp.float32),
            pltpu.VMEM((16 * _PERIOD,), jnp.float32),
            pltpu.SemaphoreType.DMA,
        ],
    )
    return f(x, quantiles, probs, initial_means, initial_stds)


def kernel(x, quantiles, probs, initial_means, initial_stds):
    return _qnorm(x, quantiles, probs, initial_means, initial_stds)


# speedup vs baseline: 1.7521x; 1.7521x over previous
import jax
import jax.numpy as jnp
from jax import lax
from jax.experimental import pallas as pl
from jax.experimental.pallas import tpu as pltpu
from jax.experimental.pallas import tpu_sc as plsc

_K = 99
_PAD_K = 128
_EPS = 1e-05
_D = 26
_B = 16384
_NW = 32
_ROWS = _B // _NW
_VPD = _ROWS // 16
_VECS = _D * _VPD


def _body(x_hbm, q_hbm, p_hbm, m_hbm, s_hbm, out_hbm,
          x_v, o_v, q_v, p_v, m_v, s_v, qpad_v, is_v, es_v, sem):
    wid = lax.axis_index("s") * 2 + lax.axis_index("c")
    rbase = wid * _ROWS

    xcopy = pltpu.async_copy(x_hbm.at[pl.ds(rbase, _ROWS), :], x_v, sem)
    pltpu.sync_copy(q_hbm, q_v)
    pltpu.sync_copy(p_hbm, p_v)
    pltpu.sync_copy(m_hbm, m_v)
    pltpu.sync_copy(s_hbm, s_v)

    @plsc.parallel_loop(0, _D * _PAD_K // 16, step=1, unroll=4)
    def build(j):
        flat = j * 16 + lax.iota(jnp.int32, 16)
        d = lax.shift_right_logical(flat, 7)
        c = lax.bitwise_and(flat, _PAD_K - 1)
        cc = jnp.minimum(c, _K - 1)
        qv = plsc.load_gather(q_v, [d, cc])
        sv = plsc.load_gather(s_v, [d])
        mv = plsc.load_gather(m_v, [d])
        qpad_v[pl.ds(j * 16, 16)] = jnp.where(c > _K - 1, jnp.inf, qv * sv + mv)

    for t in range(2):
        dd = jnp.minimum(lax.iota(jnp.int32, 16) + t * 16, _D - 1)
        sv = plsc.load_gather(s_v, [dd])
        is_v[pl.ds(t * 16, 16)] = 1.0 / sv
        es_v[pl.ds(t * 16, 16)] = _EPS * sv

    xcopy.wait()

    iota16 = lax.iota(jnp.int32, 16)

    @plsc.parallel_loop(0, _VECS, step=1, unroll=8)
    def body(v):
        d = lax.shift_right_logical(v, 5)
        roff = lax.bitwise_and(v, _VPD - 1) * 16
        qb = d * _PAD_K
        rl = roff + iota16
        cl = jnp.broadcast_to(d, (16,))
        iv = plsc.load_gather(is_v, [cl])
        es = plsc.load_gather(es_v, [cl])
        xv = plsc.load_gather(x_v, [rl, cl])

        apos = jnp.broadcast_to(qb, (16,))
        for step in (64, 32, 16, 8, 4, 2, 1):
            qv = plsc.load_gather(qpad_v, [apos + (step - 1)])
            apos = jnp.where(qv < xv, apos + step, apos)
        idx = apos - qb

        left = jnp.maximum(idx - 1, 0)
        right = jnp.minimum(idx, _K - 1)
        qls = plsc.load_gather(qpad_v, [qb + left])
        qrs = plsc.load_gather(qpad_v, [qb + right])
        pL = plsc.load_gather(p_v, [left])
        pR = plsc.load_gather(p_v, [right])

        res = pL + (xv - qls) * (pR - pL) / (qrs - qls + es)
        mlow = (idx == 0) & (xv < qls)
        mhigh = (idx == _K) & (xv > qrs)
        ta = jnp.where(mlow, qls - xv, xv - qrs) * iv
        e = jnp.exp(-2.0 * jnp.maximum(ta, 0.0))
        th = (1.0 - e) / (1.0 + e)
        res = jnp.where(mlow, pL - pL * th, res)
        res = jnp.where(mhigh, pR + (1.0 - pR) * th, res)
        plsc.store_scatter(o_v, [rl, cl], res)

    pltpu.sync_copy(o_v, out_hbm.at[pl.ds(rbase, _ROWS), :])


@jax.jit
def _qnorm(x, quantiles, probs, initial_means, initial_stds):
    mesh = plsc.VectorSubcoreMesh(core_axis_name="c", subcore_axis_name="s")
    f = pl.kernel(
        _body,
        out_type=jax.ShapeDtypeStruct((_B, _D), jnp.float32),
        mesh=mesh,
        compiler_params=pltpu.CompilerParams(
            needs_layout_passes=False, use_tc_tiling_on_sc=False),
        scratch_types=[
            pltpu.VMEM((_ROWS, _D), jnp.float32),
            pltpu.VMEM((_ROWS, _D), jnp.float32),
            pltpu.VMEM((_D, _K), jnp.float32),
            pltpu.VMEM((_K,), jnp.float32),
            pltpu.VMEM((_D,), jnp.float32),
            pltpu.VMEM((_D,), jnp.float32),
            pltpu.VMEM((_D * _PAD_K,), jnp.float32),
            pltpu.VMEM((32,), jnp.float32),
            pltpu.VMEM((32,), jnp.float32),
            pltpu.SemaphoreType.DMA,
        ],
    )
    return f(x, quantiles, probs, initial_means, initial_stds)


def kernel(x, quantiles, probs, initial_means, initial_stds):
    return _qnorm(x, quantiles, probs, initial_means, initial_stds)


# speedup vs baseline: 2.2771x; 1.2997x over previous
import jax
import jax.numpy as jnp
from jax import lax
from jax.experimental import pallas as pl
from jax.experimental.pallas import tpu as pltpu
from jax.experimental.pallas import tpu_sc as plsc

_K = 99
_PAD_K = 128
_STRIDE = 129
_EPS = 1e-05
_D = 26
_B = 16384
_N = _B * _D
_NW = 32
_ROWS = _B // _NW
_CHUNK = _N // _NW
_VECS = _CHUNK // 16
_PERIOD = 13


def _body(x_hbm, q_hbm, p_hbm, m_hbm, s_hbm, out_hbm,
          x_v, o_v, q_v, p_v, m_v, s_v, qpad_v, ppad_v,
          patq_v, patr_v, patc_v, pati_v, pate_v, sem):
    wid = lax.axis_index("s") * 2 + lax.axis_index("c")

    xcopy = pltpu.async_copy(
        x_hbm.at[pl.ds(wid * _ROWS, _ROWS), :], x_v, sem)
    pltpu.sync_copy(q_hbm, q_v)
    pltpu.sync_copy(p_hbm, p_v)
    pltpu.sync_copy(m_hbm, m_v)
    pltpu.sync_copy(s_hbm, s_v)

    @plsc.parallel_loop(0, _D * _PAD_K // 16, step=1, unroll=4)
    def build(j):
        flat = j * 16 + lax.iota(jnp.int32, 16)
        d = lax.shift_right_logical(flat, 7)
        c = lax.bitwise_and(flat, _PAD_K - 1)
        cc = jnp.minimum(c, _K - 1)
        addr = flat + d
        qv = plsc.load_gather(q_v, [d, cc])
        sv = plsc.load_gather(s_v, [d])
        mv = plsc.load_gather(m_v, [d])
        plsc.store_scatter(qpad_v, [addr],
                           jnp.where(c > _K - 1, jnp.inf, qv * sv + mv))
        plsc.store_scatter(ppad_v, [addr], plsc.load_gather(p_v, [cc]))

    for j in range(_PERIOD):
        lane = lax.iota(jnp.int32, 16) + (j * 16)
        dd = lane % _D
        sv = plsc.load_gather(s_v, [dd])
        patq_v[pl.ds(j * 16, 16)] = dd * _STRIDE
        patr_v[pl.ds(j * 16, 16)] = lax.div(lane, _D)
        patc_v[pl.ds(j * 16, 16)] = dd
        pati_v[pl.ds(j * 16, 16)] = 1.0 / sv
        pate_v[pl.ds(j * 16, 16)] = _EPS * sv

    xcopy.wait()

    @plsc.parallel_loop(0, _VECS, step=1, unroll=8,
                        carry=(jnp.int32(0), jnp.int32(0)))
    def body(v, c):
        poff, rb = c
        poff = pl.multiple_of(poff, 16)
        qb = patq_v[pl.ds(poff, 16)]
        rl = patr_v[pl.ds(poff, 16)] + rb
        cl = patc_v[pl.ds(poff, 16)]
        iv = pati_v[pl.ds(poff, 16)]
        es = pate_v[pl.ds(poff, 16)]
        xv = plsc.load_gather(x_v, [rl, cl])

        apos = qb
        for step in (64, 32, 16, 8, 4, 2, 1):
            qv = plsc.load_gather(qpad_v, [apos + (step - 1)])
            apos = jnp.where(qv < xv, apos + step, apos)
        idx = apos - qb

        left = jnp.maximum(idx - 1, 0)
        right = jnp.minimum(idx, _K - 1)
        qls = plsc.load_gather(qpad_v, [qb + left])
        qrs = plsc.load_gather(qpad_v, [qb + right])
        pL = plsc.load_gather(ppad_v, [qb + left])
        pR = plsc.load_gather(ppad_v, [qb + right])

        res = pL + (xv - qls) * (pR - pL) / (qrs - qls + es)
        mlow = (idx == 0) & (xv < qls)
        mhigh = (idx == _K) & (xv > qrs)
        ta = jnp.where(mlow, qls - xv, xv - qrs) * iv
        e = jnp.exp(-2.0 * jnp.maximum(ta, 0.0))
        th = (1.0 - e) / (1.0 + e)
        res = jnp.where(mlow, pL - pL * th, res)
        res = jnp.where(mhigh, pR + (1.0 - pR) * th, res)
        plsc.store_scatter(o_v, [rl, cl], res)
        wrap = poff == (_PERIOD - 1) * 16
        return (jnp.where(wrap, 0, poff + 16), jnp.where(wrap, rb + 8, rb))

    pltpu.sync_copy(o_v, out_hbm.at[pl.ds(wid * _ROWS, _ROWS), :])


@jax.jit
def _qnorm(x, quantiles, probs, initial_means, initial_stds):
    mesh = plsc.VectorSubcoreMesh(core_axis_name="c", subcore_axis_name="s")
    f = pl.kernel(
        _body,
        out_type=jax.ShapeDtypeStruct((_B, _D), jnp.float32),
        mesh=mesh,
        compiler_params=pltpu.CompilerParams(
            needs_layout_passes=False, use_tc_tiling_on_sc=False),
        scratch_types=[
            pltpu.VMEM((_ROWS, _D), jnp.float32),
            pltpu.VMEM((_ROWS, _D), jnp.float32),
            pltpu.VMEM((_D, _K), jnp.float32),
            pltpu.VMEM((_K,), jnp.float32),
            pltpu.VMEM((_D,), jnp.float32),
            pltpu.VMEM((_D,), jnp.float32),
            pltpu.VMEM((_D * _STRIDE,), jnp.float32),
            pltpu.VMEM((_D * _STRIDE,), jnp.float32),
            pltpu.VMEM((16 * _PERIOD,), jnp.int32),
            pltpu.VMEM((16 * _PERIOD,), jnp.int32),
            pltpu.VMEM((16 * _PERIOD,), jnp.int32),
            pltpu.VMEM((16 * _PERIOD,), jnp.float32),
            pltpu.VMEM((16 * _PERIOD,), jnp.float32),
            pltpu.SemaphoreType.DMA,
        ],
    )
    return f(x, quantiles, probs, initial_means, initial_stds)


def kernel(x, quantiles, probs, initial_means, initial_stds):
    return _qnorm(x, quantiles, probs, initial_means, initial_stds)


# speedup vs baseline: 2.4558x; 1.0785x over previous
import jax
import jax.numpy as jnp
from jax import lax
from jax.experimental import pallas as pl
from jax.experimental.pallas import tpu as pltpu
from jax.experimental.pallas import tpu_sc as plsc

_K = 99
_PAD_K = 128
_STRIDE = 129
_EPS = 1e-05
_D = 26
_DP = 32
_B = 16384
_NW = 32
_ROWS = _B // _NW


def _body(x_hbm, q_hbm, p_hbm, m_hbm, s_hbm, out_hbm,
          x_v, o_v, q_v, p_v, m_v, s_v, qpad_v, ppad_v, sem):
    wid = lax.axis_index("s") * 2 + lax.axis_index("c")
    rbase = wid * _ROWS

    xcopy = pltpu.async_copy(x_hbm.at[pl.ds(rbase, _ROWS), :], x_v, sem)
    pltpu.sync_copy(q_hbm, q_v)
    pltpu.sync_copy(p_hbm, p_v)
    pltpu.sync_copy(m_hbm, m_v)
    pltpu.sync_copy(s_hbm, s_v)

    @plsc.parallel_loop(0, _D * _PAD_K // 16, step=1, unroll=4)
    def build(j):
        flat = j * 16 + lax.iota(jnp.int32, 16)
        d = lax.shift_right_logical(flat, 7)
        c = lax.bitwise_and(flat, _PAD_K - 1)
        cc = jnp.minimum(c, _K - 1)
        addr = flat + d
        qv = plsc.load_gather(q_v, [d, cc])
        sv = plsc.load_gather(s_v, [d])
        mv = plsc.load_gather(m_v, [d])
        plsc.store_scatter(qpad_v, [addr],
                           jnp.where(c > _K - 1, jnp.inf, qv * sv + mv))
        plsc.store_scatter(ppad_v, [addr], plsc.load_gather(p_v, [cc]))

    iota16 = lax.iota(jnp.int32, 16)
    d0 = iota16
    d1 = (iota16 + 16) % _D
    sv0 = plsc.load_gather(s_v, [d0])
    sv1 = plsc.load_gather(s_v, [d1])
    halves = (
        (0, d0 * _STRIDE, 1.0 / sv0, _EPS * sv0),
        (16, d1 * _STRIDE, 1.0 / sv1, _EPS * sv1),
    )

    xcopy.wait()

    @plsc.parallel_loop(0, _ROWS, step=1, unroll=4)
    def body(r):
        for (col, qb, iv, es) in halves:
            xv = x_v[r, pl.ds(col, 16)]

            apos = qb
            for step in (64, 32, 16, 8, 4, 2, 1):
                qv = plsc.load_gather(qpad_v, [apos + (step - 1)])
                apos = jnp.where(qv < xv, apos + step, apos)
            idx = apos - qb

            left = jnp.maximum(idx - 1, 0)
            right = jnp.minimum(idx, _K - 1)
            qls = plsc.load_gather(qpad_v, [qb + left])
            qrs = plsc.load_gather(qpad_v, [qb + right])
            pL = plsc.load_gather(ppad_v, [qb + left])
            pR = plsc.load_gather(ppad_v, [qb + right])

            res = pL + (xv - qls) * (pR - pL) / (qrs - qls + es)
            mlow = (idx == 0) & (xv < qls)
            mhigh = (idx == _K) & (xv > qrs)
            ta = jnp.where(mlow, qls - xv, xv - qrs) * iv
            e = jnp.exp(-2.0 * jnp.maximum(ta, 0.0))
            th = (1.0 - e) / (1.0 + e)
            res = jnp.where(mlow, pL - pL * th, res)
            res = jnp.where(mhigh, pR + (1.0 - pR) * th, res)
            o_v[r, pl.ds(col, 16)] = res

    pltpu.sync_copy(o_v, out_hbm.at[pl.ds(rbase, _ROWS), :])


@jax.jit
def _qnorm(xp, quantiles, probs, initial_means, initial_stds):
    mesh = plsc.VectorSubcoreMesh(core_axis_name="c", subcore_axis_name="s")
    f = pl.kernel(
        _body,
        out_type=jax.ShapeDtypeStruct((_B, _DP), jnp.float32),
        mesh=mesh,
        compiler_params=pltpu.CompilerParams(
            needs_layout_passes=False, use_tc_tiling_on_sc=False),
        scratch_types=[
            pltpu.VMEM((_ROWS, _DP), jnp.float32),
            pltpu.VMEM((_ROWS, _DP), jnp.float32),
            pltpu.VMEM((_D, _K), jnp.float32),
            pltpu.VMEM((_K,), jnp.float32),
            pltpu.VMEM((_D,), jnp.float32),
            pltpu.VMEM((_D,), jnp.float32),
            pltpu.VMEM((_D * _STRIDE,), jnp.float32),
            pltpu.VMEM((_D * _STRIDE,), jnp.float32),
            pltpu.SemaphoreType.DMA,
        ],
    )
    return f(xp, quantiles, probs, initial_means, initial_stds)


def kernel(x, quantiles, probs, initial_means, initial_stds):
    xp = jnp.pad(x, ((0, 0), (0, _DP - _D)))
    out = _qnorm(xp, quantiles, probs, initial_means, initial_stds)
    return out[:, :_D]
